# baseline (speedup 1.0000x reference)
import functools

import jax
import jax.numpy as jnp
from jax import lax
from jax.experimental import pallas as pl
from jax.experimental.pallas import tpu as pltpu
from jax.experimental.pallas import tpu_sc as plsc

EMB = 64
G = 50
G_PAD = 64
L_ITEMS = 200
ITEMS_PER_WORKER = 8
N_ITEM_WORKERS = L_ITEMS // ITEMS_PER_WORKER
LANES = 16

_info = plsc.get_sparse_core_info()
_NC = _info.num_cores

_mesh = plsc.VectorSubcoreMesh(core_axis_name="c", subcore_axis_name="s")


@functools.partial(
    pl.kernel,
    mesh=_mesh,
    compiler_params=pltpu.CompilerParams(use_tc_tiling_on_sc=False),
    out_type=jax.ShapeDtypeStruct((L_ITEMS + 1, EMB), jnp.float32),
    scratch_types=[
        pltpu.VMEM((ITEMS_PER_WORKER,), jnp.int32),
        pltpu.VMEM((ITEMS_PER_WORKER, EMB), jnp.float32),
        pltpu.VMEM((G_PAD,), jnp.int32),
        pltpu.VMEM((G_PAD, EMB), jnp.float32),
        pltpu.VMEM((1, EMB), jnp.float32),
        pltpu.SemaphoreType.DMA,
    ],
)
def _embed_sc(gm_hbm, hist_hbm, user_hbm, item_hbm, out_hbm,
              idx_i, rows_i, idx_u, rows_u, sum_v, sem):
    wid = lax.axis_index("s") * _NC + lax.axis_index("c")

    @pl.when(wid < N_ITEM_WORKERS)
    def _items():
        base = wid * ITEMS_PER_WORKER
        pltpu.sync_copy(hist_hbm.at[pl.ds(base, ITEMS_PER_WORKER)], idx_i)
        pltpu.async_copy(item_hbm.at[idx_i], rows_i, sem).wait()
        pltpu.sync_copy(rows_i, out_hbm.at[pl.ds(base, ITEMS_PER_WORKER)])

    @pl.when(wid == N_ITEM_WORKERS)
    def _users():
        pltpu.sync_copy(gm_hbm, idx_u)
        pltpu.async_copy(user_hbm.at[idx_u], rows_u, sem).wait()
        for c in range(EMB // LANES):
            acc = rows_u[0, pl.ds(c * LANES, LANES)]
            for r in range(1, G):
                acc = acc + rows_u[r, pl.ds(c * LANES, LANES)]
            sum_v[0, pl.ds(c * LANES, LANES)] = acc
        pltpu.sync_copy(sum_v, out_hbm.at[pl.ds(L_ITEMS, 1)])


def kernel(group_members, history, user_table, item_table, W1, b1, W2, b2):
    gm_pad = jnp.concatenate(
        [group_members, jnp.zeros((G_PAD - G,), dtype=group_members.dtype)])
    out = _embed_sc(gm_pad, history, user_table, item_table)
    return jnp.concatenate([out[L_ITEMS], out[:L_ITEMS].reshape(-1)])


# speedup vs baseline: 1.6661x; 1.6661x over previous
import functools

import jax
import jax.numpy as jnp
from jax import lax
from jax.experimental import pallas as pl
from jax.experimental.pallas import tpu as pltpu
from jax.experimental.pallas import tpu_sc as plsc

EMB = 64
G = 50
G_PAD = 64
L_ITEMS = 200
ITEMS_PER_WORKER = 8
N_ITEM_WORKERS = L_ITEMS // ITEMS_PER_WORKER
LANES = 16

_info = plsc.get_sparse_core_info()
_NC = _info.num_cores

_mesh = plsc.VectorSubcoreMesh(core_axis_name="c", subcore_axis_name="s")


@functools.partial(
    pl.kernel,
    mesh=_mesh,
    out_type=jax.ShapeDtypeStruct((L_ITEMS + 1, EMB), jnp.float32),
    scratch_types=[
        pltpu.VMEM((LANES,), jnp.int32),
        pltpu.VMEM((ITEMS_PER_WORKER, EMB), jnp.float32),
        pltpu.VMEM((G_PAD,), jnp.int32),
        pltpu.VMEM((G_PAD, EMB), jnp.float32),
        pltpu.VMEM((1, EMB), jnp.float32),
        pltpu.SemaphoreType.DMA,
    ],
)
def _embed_sc(gm_hbm, hist_hbm, user_hbm, item_hbm, out_hbm,
              idx_i, rows_i, idx_u, rows_u, sum_v, sem):
    wid = lax.axis_index("s") * _NC + lax.axis_index("c")

    @pl.when(wid < N_ITEM_WORKERS)
    def _items():
        base = wid * ITEMS_PER_WORKER
        pltpu.sync_copy(hist_hbm.at[pl.ds(base, ITEMS_PER_WORKER)],
                        idx_i.at[pl.ds(0, ITEMS_PER_WORKER)])
        iv = idx_i[...]
        for j in range(ITEMS_PER_WORKER):
            row = iv[j]
            pltpu.async_copy(item_hbm.at[pl.ds(row, 1)],
                             rows_i.at[pl.ds(j, 1)], sem)
        pltpu.make_async_copy(item_hbm.at[pl.ds(0, ITEMS_PER_WORKER)],
                              rows_i, sem).wait()
        pltpu.sync_copy(rows_i, out_hbm.at[pl.ds(base, ITEMS_PER_WORKER)])

    @pl.when(wid == N_ITEM_WORKERS)
    def _users():
        pltpu.sync_copy(gm_hbm, idx_u)
        for c in range(G_PAD // LANES):
            uv = idx_u[pl.ds(c * LANES, LANES)]
            for k in range(LANES):
                r = c * LANES + k
                pltpu.async_copy(user_hbm.at[pl.ds(uv[k], 1)],
                                 rows_u.at[pl.ds(r, 1)], sem)
        pltpu.make_async_copy(user_hbm.at[pl.ds(0, G_PAD)],
                              rows_u, sem).wait()
        for c in range(EMB // LANES):
            acc = rows_u[0, pl.ds(c * LANES, LANES)]
            for r in range(1, G):
                acc = acc + rows_u[r, pl.ds(c * LANES, LANES)]
            sum_v[0, pl.ds(c * LANES, LANES)] = acc
        pltpu.sync_copy(sum_v, out_hbm.at[pl.ds(L_ITEMS, 1)])


def kernel(group_members, history, user_table, item_table, W1, b1, W2, b2):
    gm_pad = jnp.concatenate(
        [group_members, jnp.zeros((G_PAD - G,), dtype=group_members.dtype)])
    out = _embed_sc(gm_pad, history, user_table, item_table)
    return jnp.concatenate([out[L_ITEMS], out[:L_ITEMS].reshape(-1)])


# speedup vs baseline: 1.6774x; 1.0068x over previous
import functools

import jax
import jax.numpy as jnp
from jax import lax
from jax.experimental import pallas as pl
from jax.experimental.pallas import tpu as pltpu
from jax.experimental.pallas import tpu_sc as plsc

EMB = 64
G = 50
G_PAD = 64
L_ITEMS = 200
ITEMS_PER_WORKER = 8
N_ITEM_WORKERS = L_ITEMS // ITEMS_PER_WORKER
LANES = 16

_info = plsc.get_sparse_core_info()
_NC = _info.num_cores

_mesh = plsc.VectorSubcoreMesh(core_axis_name="c", subcore_axis_name="s")


@functools.partial(
    pl.kernel,
    mesh=_mesh,
    out_type=jax.ShapeDtypeStruct((L_ITEMS + 1, EMB), jnp.float32),
    scratch_types=[
        pltpu.VMEM((LANES,), jnp.int32),
        pltpu.VMEM((ITEMS_PER_WORKER, EMB), jnp.float32),
        pltpu.VMEM((G_PAD,), jnp.int32),
        pltpu.VMEM((G_PAD, EMB), jnp.float32),
        pltpu.VMEM((1, EMB), jnp.float32),
        pltpu.SemaphoreType.DMA,
    ],
)
def _embed_sc(gm_hbm, hist_hbm, user_hbm, item_hbm, out_hbm,
              idx_i, rows_i, idx_u, rows_u, sum_v, sem):
    wid = lax.axis_index("s") * _NC + lax.axis_index("c")

    @pl.when(wid < N_ITEM_WORKERS)
    def _items():
        base = wid * ITEMS_PER_WORKER
        pltpu.sync_copy(hist_hbm.at[pl.ds(base, ITEMS_PER_WORKER)],
                        idx_i.at[pl.ds(0, ITEMS_PER_WORKER)])
        iv = idx_i[...]
        for j in range(ITEMS_PER_WORKER):
            pltpu.async_copy(item_hbm.at[pl.ds(iv[j], 1)],
                             rows_i.at[pl.ds(j, 1)], sem)
        pltpu.make_async_copy(item_hbm.at[pl.ds(0, ITEMS_PER_WORKER)],
                              rows_i, sem).wait()
        pltpu.sync_copy(rows_i, out_hbm.at[pl.ds(base, ITEMS_PER_WORKER)])

    @pl.when(wid == N_ITEM_WORKERS)
    def _users():
        pltpu.sync_copy(gm_hbm, idx_u)

        def issue_chunk(c, carry):
            uv = idx_u[pl.ds(c * LANES, LANES)]
            for k in range(LANES):
                pltpu.async_copy(user_hbm.at[pl.ds(uv[k], 1)],
                                 rows_u.at[pl.ds(c * LANES + k, 1)], sem)
            return carry

        lax.fori_loop(0, G_PAD // LANES, issue_chunk, 0)
        pltpu.make_async_copy(user_hbm.at[pl.ds(0, G_PAD)],
                              rows_u, sem).wait()

        zero = jnp.zeros((LANES,), jnp.float32)

        def sum_row(r, accs):
            return tuple(
                accs[c] + rows_u[r, pl.ds(c * LANES, LANES)]
                for c in range(EMB // LANES))

        accs = lax.fori_loop(0, G, sum_row, (zero,) * (EMB // LANES))
        for c in range(EMB // LANES):
            sum_v[0, pl.ds(c * LANES, LANES)] = accs[c]
        pltpu.sync_copy(sum_v, out_hbm.at[pl.ds(L_ITEMS, 1)])


def kernel(group_members, history, user_table, item_table, W1, b1, W2, b2):
    gm_pad = jnp.concatenate(
        [group_members, jnp.zeros((G_PAD - G,), dtype=group_members.dtype)])
    out = _embed_sc(gm_pad, history, user_table, item_table)
    return jnp.concatenate([out[L_ITEMS], out[:L_ITEMS].reshape(-1)])


# speedup vs baseline: 24.9138x; 14.8528x over previous
import functools

import jax
import jax.numpy as jnp
from jax import lax
from jax.experimental import pallas as pl
from jax.experimental.pallas import tpu as pltpu
from jax.experimental.pallas import tpu_sc as plsc

EMB = 64
G = 50
G_PAD = 56
L_ITEMS = 200
PER_W = 8
N_ITEM_WORKERS = L_ITEMS // PER_W
N_USER_WORKERS = G_PAD // PER_W
LANES = 16
BLK = 128

_info = plsc.get_sparse_core_info()
_NC = _info.num_cores

_mesh = plsc.VectorSubcoreMesh(core_axis_name="c", subcore_axis_name="s")


@functools.partial(
    pl.kernel,
    mesh=_mesh,
    compiler_params=pltpu.CompilerParams(needs_layout_passes=False),
    out_type=(
        jax.ShapeDtypeStruct((L_ITEMS, EMB), jnp.float32),
        jax.ShapeDtypeStruct((N_USER_WORKERS + 1, EMB), jnp.float32),
    ),
    scratch_types=[
        pltpu.VMEM((LANES,), jnp.int32),
        pltpu.VMEM((PER_W, EMB, BLK), jnp.float32),
        pltpu.VMEM((PER_W, EMB), jnp.float32),
        pltpu.SemaphoreType.DMA,
    ],
)
def _embed_sc(gm_hbm, hist_hbm, user_t, item_t, out_items, out_gsum,
              idx_v, blocks, rows, sem):
    wid = lax.axis_index("s") * _NC + lax.axis_index("c")
    dvecs = [c * LANES + lax.iota(jnp.int32, LANES)
             for c in range(EMB // LANES)]

    def fetch(table_t, iv):
        lanes = []
        for j in range(PER_W):
            val = iv[j]
            base = pl.multiple_of((val >> 7) * BLK, BLK)
            pltpu.async_copy(table_t.at[:, pl.ds(base, BLK)],
                             blocks.at[j], sem)
            lanes.append(val & (BLK - 1))
        for j in range(PER_W):
            pltpu.make_async_copy(table_t.at[:, pl.ds(0, BLK)],
                                  blocks.at[j], sem).wait()
        return lanes

    @pl.when(wid < N_ITEM_WORKERS)
    def _items():
        base = wid * PER_W
        pltpu.sync_copy(hist_hbm.at[pl.ds(base, PER_W)],
                        idx_v.at[pl.ds(0, PER_W)])
        lanes = fetch(item_t, idx_v[...])
        for j in range(PER_W):
            jc = jnp.full((LANES,), j, jnp.int32)
            lc = jnp.full((LANES,), lanes[j], jnp.int32)
            for c in range(EMB // LANES):
                rows[j, pl.ds(c * LANES, LANES)] = plsc.load_gather(
                    blocks, [jc, dvecs[c], lc])
        pltpu.sync_copy(rows, out_items.at[pl.ds(base, PER_W)])

    @pl.when(wid >= N_ITEM_WORKERS)
    def _users():
        uw = wid - N_ITEM_WORKERS
        ubase = uw * PER_W
        pltpu.sync_copy(gm_hbm.at[pl.ds(ubase, PER_W)],
                        idx_v.at[pl.ds(0, PER_W)])
        lanes = fetch(user_t, idx_v[...])
        accs = [jnp.zeros((LANES,), jnp.float32)
                for _ in range(EMB // LANES)]
        for j in range(PER_W):
            jc = jnp.full((LANES,), j, jnp.int32)
            lc = jnp.full((LANES,), lanes[j], jnp.int32)
            valid = (ubase + j) < G
            vm = jnp.full((LANES,), valid)
            for c in range(EMB // LANES):
                g = plsc.load_gather(blocks, [jc, dvecs[c], lc])
                accs[c] = accs[c] + jnp.where(vm, g, 0.0)
        for c in range(EMB // LANES):
            rows[0, pl.ds(c * LANES, LANES)] = accs[c]
        pltpu.sync_copy(rows.at[pl.ds(0, 1)], out_gsum.at[pl.ds(uw, 1)])


def kernel(group_members, history, user_table, item_table, W1, b1, W2, b2):
    gm_pad = jnp.concatenate(
        [group_members, jnp.zeros((G_PAD - G,), dtype=group_members.dtype)])
    out_items, out_gsum = _embed_sc(
        gm_pad, history, user_table.T, item_table.T)
    group = out_gsum[:N_USER_WORKERS].sum(axis=0)
    return jnp.concatenate([group, out_items.reshape(-1)])
